# Initial kernel scaffold; baseline (speedup 1.0000x reference)
import jax
import jax.numpy as jnp
from jax.experimental import pallas as pl


def kernel(coordinates, edge_src, edge_dst):
    raise NotImplementedError("write your pallas kernel here")


# speedup vs baseline: 9.6200x; 9.6200x over previous
import functools

import jax
import jax.numpy as jnp
from jax import lax
from jax.experimental import pallas as pl
from jax.experimental.pallas import tpu as pltpu
from jax.experimental.pallas import tpu_sc as plsc

_CUTOFF = 5.0
_N_NODES = 100000
_N_EDGES = 6400000

_LANES = 16
_NW = 32
_EPT = _N_EDGES // _NW
_K = 2000
_NCH = _EPT // _K
_NIT = _K // _LANES

_PI_OVER_CUTOFF = 0.6283185307179586
_HALF_PI = 1.5707963267948966


def _edge_body(x_hbm, y_hbm, z_hbm, src_hbm, dst_hbm,
               vec_hbm, dist_hbm, sw_hbm, dxp_hbm, dyp_hbm,
               table, srcb, dstb, dxb, dyb, vecb, distb, swb, sem):
    cid = lax.axis_index("c")
    sid = lax.axis_index("s")
    wid = sid * 2 + cid
    ebase = wid * _EPT

    lane = lax.iota(jnp.int32, _LANES)

    def diff_pass(col_hbm, out_hbm):
        pltpu.sync_copy(col_hbm, table)

        def chunk(s, carry):
            base = ebase + s * _K
            pltpu.sync_copy(src_hbm.at[pl.ds(base, _K)], srcb)
            pltpu.sync_copy(dst_hbm.at[pl.ds(base, _K)], dstb)

            def inner(i, _):
                o = i * _LANES
                isrc = srcb[pl.ds(o, _LANES)]
                idst = dstb[pl.ds(o, _LANES)]
                cs = plsc.load_gather(table, [isrc])
                cd = plsc.load_gather(table, [idst])
                dxb[pl.ds(o, _LANES)] = cd - cs
                return _

            lax.fori_loop(0, _NIT, inner, None)
            pltpu.sync_copy(dxb, out_hbm.at[pl.ds(base, _K)])
            return carry

        lax.fori_loop(0, _NCH, chunk, None)

    diff_pass(x_hbm, dxp_hbm)
    diff_pass(y_hbm, dyp_hbm)

    pltpu.sync_copy(z_hbm, table)

    def chunk_z(s, carry):
        base = ebase + s * _K
        pltpu.sync_copy(src_hbm.at[pl.ds(base, _K)], srcb)
        pltpu.sync_copy(dst_hbm.at[pl.ds(base, _K)], dstb)
        pltpu.sync_copy(dxp_hbm.at[pl.ds(base, _K)], dxb)
        pltpu.sync_copy(dyp_hbm.at[pl.ds(base, _K)], dyb)

        def inner(i, _):
            o = i * _LANES
            isrc = srcb[pl.ds(o, _LANES)]
            idst = dstb[pl.ds(o, _LANES)]
            zs = plsc.load_gather(table, [isrc])
            zd = plsc.load_gather(table, [idst])
            dz = zd - zs
            dx = dxb[pl.ds(o, _LANES)]
            dy = dyb[pl.ds(o, _LANES)]
            d2 = jnp.maximum(dx * dx + dy * dy + dz * dz, 1e-30)
            iy = jnp.int32(0x5F3759DF) - (plsc.bitcast(d2, jnp.int32) >> 1)
            y = plsc.bitcast(iy, jnp.float32)
            y = y * (1.5 - 0.5 * d2 * y * y)
            y = y * (1.5 - 0.5 * d2 * y * y)
            y = y * (1.5 - 0.5 * d2 * y * y)
            dist = d2 * y
            p = dist * _PI_OVER_CUTOFF - _HALF_PI
            z2 = p * p
            s_ = p * (1.0 + z2 * (-1.6666667e-01 + z2 * (8.3333333e-03
                 + z2 * (-1.9841270e-04 + z2 * 2.7557319e-06))))
            sw = jnp.where(dist < _CUTOFF, 0.5 - 0.5 * s_, 0.0)
            e3 = (o + lane) * 3
            plsc.store_scatter(vecb, [e3], dx)
            plsc.store_scatter(vecb, [e3 + 1], dy)
            plsc.store_scatter(vecb, [e3 + 2], dz)
            distb[pl.ds(o, _LANES)] = dist
            swb[pl.ds(o, _LANES)] = sw
            return _

        lax.fori_loop(0, _NIT, inner, None)
        pltpu.sync_copy(vecb, vec_hbm.at[pl.ds(base * 3, _K * 3)])
        pltpu.sync_copy(distb, dist_hbm.at[pl.ds(base, _K)])
        pltpu.sync_copy(swb, sw_hbm.at[pl.ds(base, _K)])
        return carry

    lax.fori_loop(0, _NCH, chunk_z, None)


@functools.partial(jax.jit, donate_argnums=())
def _run(xcol, ycol, zcol, src, dst):
    mesh = plsc.VectorSubcoreMesh(core_axis_name="c", subcore_axis_name="s")
    f = pl.kernel(
        _edge_body,
        mesh=mesh,
        compiler_params=pltpu.CompilerParams(needs_layout_passes=False),
        out_type=(
            jax.ShapeDtypeStruct((_N_EDGES * 3,), jnp.float32),
            jax.ShapeDtypeStruct((_N_EDGES,), jnp.float32),
            jax.ShapeDtypeStruct((_N_EDGES,), jnp.float32),
            jax.ShapeDtypeStruct((_N_EDGES,), jnp.float32),
            jax.ShapeDtypeStruct((_N_EDGES,), jnp.float32),
        ),
        scratch_types=[
            pltpu.VMEM((_N_NODES,), jnp.float32),
            pltpu.VMEM((_K,), jnp.int32),
            pltpu.VMEM((_K,), jnp.int32),
            pltpu.VMEM((_K,), jnp.float32),
            pltpu.VMEM((_K,), jnp.float32),
            pltpu.VMEM((_K * 3,), jnp.float32),
            pltpu.VMEM((_K,), jnp.float32),
            pltpu.VMEM((_K,), jnp.float32),
            pltpu.SemaphoreType.DMA,
        ],
    )
    return f(xcol, ycol, zcol, src, dst)


def kernel(coordinates, edge_src, edge_dst):
    xcol = coordinates[:, 0]
    ycol = coordinates[:, 1]
    zcol = coordinates[:, 2]
    vecf, distances, switch, _, _ = _run(xcol, ycol, zcol, edge_src, edge_dst)
    vec = vecf.reshape(_N_EDGES, 3)
    edge_mask = distances < _CUTOFF
    return (vec, distances, switch, edge_mask)


# speedup vs baseline: 10.5620x; 1.0979x over previous
import functools

import jax
import jax.numpy as jnp
from jax import lax
from jax.experimental import pallas as pl
from jax.experimental.pallas import tpu as pltpu
from jax.experimental.pallas import tpu_sc as plsc

_CUTOFF = 5.0
_N_NODES = 100000
_N_EDGES = 6400000

_LANES = 16
_NW = 32
_EPT = _N_EDGES // _NW
_K = 2000
_NCH = _EPT // _K
_NIT = _K // _LANES

_PI_OVER_CUTOFF = 0.6283185307179586
_HALF_PI = 1.5707963267948966


def _edge_body(x_hbm, y_hbm, z_hbm, src_hbm, dst_hbm,
               vec_hbm, dist_hbm, sw_hbm, dxp_hbm, dyp_hbm,
               table, srcb, dstb, dxb, dyb, vecb, distb, swb, sem):
    cid = lax.axis_index("c")
    sid = lax.axis_index("s")
    wid = sid * 2 + cid
    ebase = wid * _EPT

    lane = lax.iota(jnp.int32, _LANES)

    def diff_pass(col_hbm, out_hbm):
        pltpu.sync_copy(col_hbm, table)

        def chunk(s, carry):
            base = ebase + s * _K
            pltpu.sync_copy(src_hbm.at[pl.ds(base, _K)], srcb)
            pltpu.sync_copy(dst_hbm.at[pl.ds(base, _K)], dstb)

            @plsc.parallel_loop(0, _K, step=_LANES, unroll=8)
            def inner(o):
                isrc = srcb[pl.ds(o, _LANES)]
                idst = dstb[pl.ds(o, _LANES)]
                cs = plsc.load_gather(table, [isrc])
                cd = plsc.load_gather(table, [idst])
                dxb[pl.ds(o, _LANES)] = cd - cs
            pltpu.sync_copy(dxb, out_hbm.at[pl.ds(base, _K)])
            return carry

        lax.fori_loop(0, _NCH, chunk, None)

    diff_pass(x_hbm, dxp_hbm)
    diff_pass(y_hbm, dyp_hbm)

    pltpu.sync_copy(z_hbm, table)

    def chunk_z(s, carry):
        base = ebase + s * _K
        pltpu.sync_copy(src_hbm.at[pl.ds(base, _K)], srcb)
        pltpu.sync_copy(dst_hbm.at[pl.ds(base, _K)], dstb)
        pltpu.sync_copy(dxp_hbm.at[pl.ds(base, _K)], dxb)
        pltpu.sync_copy(dyp_hbm.at[pl.ds(base, _K)], dyb)

        @plsc.parallel_loop(0, _K, step=_LANES, unroll=8)
        def inner(o):
            isrc = srcb[pl.ds(o, _LANES)]
            idst = dstb[pl.ds(o, _LANES)]
            zs = plsc.load_gather(table, [isrc])
            zd = plsc.load_gather(table, [idst])
            dz = zd - zs
            dx = dxb[pl.ds(o, _LANES)]
            dy = dyb[pl.ds(o, _LANES)]
            d2 = jnp.maximum(dx * dx + dy * dy + dz * dz, 1e-30)
            iy = jnp.int32(0x5F3759DF) - (plsc.bitcast(d2, jnp.int32) >> 1)
            y = plsc.bitcast(iy, jnp.float32)
            y = y * (1.5 - 0.5 * d2 * y * y)
            y = y * (1.5 - 0.5 * d2 * y * y)
            y = y * (1.5 - 0.5 * d2 * y * y)
            dist = d2 * y
            p = dist * _PI_OVER_CUTOFF - _HALF_PI
            z2 = p * p
            s_ = p * (1.0 + z2 * (-1.6666667e-01 + z2 * (8.3333333e-03
                 + z2 * (-1.9841270e-04 + z2 * 2.7557319e-06))))
            sw = jnp.where(dist < _CUTOFF, 0.5 - 0.5 * s_, 0.0)
            e3 = (o + lane) * 3
            plsc.store_scatter(vecb, [e3], dx)
            plsc.store_scatter(vecb, [e3 + 1], dy)
            plsc.store_scatter(vecb, [e3 + 2], dz)
            distb[pl.ds(o, _LANES)] = dist
            swb[pl.ds(o, _LANES)] = sw

        pltpu.sync_copy(vecb, vec_hbm.at[pl.ds(base * 3, _K * 3)])
        pltpu.sync_copy(distb, dist_hbm.at[pl.ds(base, _K)])
        pltpu.sync_copy(swb, sw_hbm.at[pl.ds(base, _K)])
        return carry

    lax.fori_loop(0, _NCH, chunk_z, None)


@functools.partial(jax.jit, donate_argnums=())
def _run(xcol, ycol, zcol, src, dst):
    mesh = plsc.VectorSubcoreMesh(core_axis_name="c", subcore_axis_name="s")
    f = pl.kernel(
        _edge_body,
        mesh=mesh,
        compiler_params=pltpu.CompilerParams(needs_layout_passes=False),
        out_type=(
            jax.ShapeDtypeStruct((_N_EDGES * 3,), jnp.float32),
            jax.ShapeDtypeStruct((_N_EDGES,), jnp.float32),
            jax.ShapeDtypeStruct((_N_EDGES,), jnp.float32),
            jax.ShapeDtypeStruct((_N_EDGES,), jnp.float32),
            jax.ShapeDtypeStruct((_N_EDGES,), jnp.float32),
        ),
        scratch_types=[
            pltpu.VMEM((_N_NODES,), jnp.float32),
            pltpu.VMEM((_K,), jnp.int32),
            pltpu.VMEM((_K,), jnp.int32),
            pltpu.VMEM((_K,), jnp.float32),
            pltpu.VMEM((_K,), jnp.float32),
            pltpu.VMEM((_K * 3,), jnp.float32),
            pltpu.VMEM((_K,), jnp.float32),
            pltpu.VMEM((_K,), jnp.float32),
            pltpu.SemaphoreType.DMA,
        ],
    )
    return f(xcol, ycol, zcol, src, dst)


def kernel(coordinates, edge_src, edge_dst):
    xcol = coordinates[:, 0]
    ycol = coordinates[:, 1]
    zcol = coordinates[:, 2]
    vecf, distances, switch, _, _ = _run(xcol, ycol, zcol, edge_src, edge_dst)
    vec = vecf.reshape(_N_EDGES, 3)
    edge_mask = distances < _CUTOFF
    return (vec, distances, switch, edge_mask)


# speedup vs baseline: 55.9776x; 5.2999x over previous
import functools

import jax
import jax.numpy as jnp
from jax import lax
from jax.experimental import pallas as pl
from jax.experimental.pallas import tpu as pltpu
from jax.experimental.pallas import tpu_sc as plsc

_CUTOFF = 5.0
_N_NODES = 100000
_N_EDGES = 6400000

_LANES = 16
_NW = 32
_EPT = _N_EDGES // _NW
_K = 2000
_NCH = _EPT // _K
_NIT = _K // _LANES

_PI_OVER_CUTOFF = 0.6283185307179586
_HALF_PI = 1.5707963267948966


def _edge_body(x_hbm, y_hbm, z_hbm, src_hbm, dst_hbm,
               vec_hbm, dist_hbm, sw_hbm, dxp_hbm, dyp_hbm,
               table, srcb, dstb, dxb, dyb, vecb, distb, swb, sem):
    cid = lax.axis_index("c")
    sid = lax.axis_index("s")
    wid = sid * 2 + cid
    ebase = wid * _EPT

    lane = lax.iota(jnp.int32, _LANES)

    def diff_pass(col_hbm, out_hbm):
        pltpu.sync_copy(col_hbm, table)

        def chunk(s, carry):
            base = ebase + s * _K
            pltpu.sync_copy(src_hbm.at[pl.ds(base, _K)], srcb)
            pltpu.sync_copy(dst_hbm.at[pl.ds(base, _K)], dstb)

            @plsc.parallel_loop(0, _K, step=_LANES, unroll=8)
            def inner(o):
                isrc = srcb[pl.ds(o, _LANES)]
                idst = dstb[pl.ds(o, _LANES)]
                cs = plsc.load_gather(table, [isrc])
                cd = plsc.load_gather(table, [idst])
                dxb[pl.ds(o, _LANES)] = cd - cs
            pltpu.sync_copy(dxb, out_hbm.at[pl.ds(base, _K)])
            return carry

        lax.fori_loop(0, _NCH, chunk, None)

    diff_pass(x_hbm, dxp_hbm)
    diff_pass(y_hbm, dyp_hbm)

    pltpu.sync_copy(z_hbm, table)

    def chunk_z(s, carry):
        base = ebase + s * _K
        pltpu.sync_copy(src_hbm.at[pl.ds(base, _K)], srcb)
        pltpu.sync_copy(dst_hbm.at[pl.ds(base, _K)], dstb)
        pltpu.sync_copy(dxp_hbm.at[pl.ds(base, _K)], dxb)
        pltpu.sync_copy(dyp_hbm.at[pl.ds(base, _K)], dyb)

        @plsc.parallel_loop(0, _K, step=_LANES, unroll=8)
        def inner(o):
            isrc = srcb[pl.ds(o, _LANES)]
            idst = dstb[pl.ds(o, _LANES)]
            zs = plsc.load_gather(table, [isrc])
            zd = plsc.load_gather(table, [idst])
            dz = zd - zs
            dx = dxb[pl.ds(o, _LANES)]
            dy = dyb[pl.ds(o, _LANES)]
            d2 = jnp.maximum(dx * dx + dy * dy + dz * dz, 1e-30)
            iy = jnp.int32(0x5F3759DF) - (plsc.bitcast(d2, jnp.int32) >> 1)
            y = plsc.bitcast(iy, jnp.float32)
            y = y * (1.5 - 0.5 * d2 * y * y)
            y = y * (1.5 - 0.5 * d2 * y * y)
            y = y * (1.5 - 0.5 * d2 * y * y)
            dist = d2 * y
            p = dist * _PI_OVER_CUTOFF - _HALF_PI
            z2 = p * p
            s_ = p * (1.0 + z2 * (-1.6666667e-01 + z2 * (8.3333333e-03
                 + z2 * (-1.9841270e-04 + z2 * 2.7557319e-06))))
            sw = jnp.where(dist < _CUTOFF, 0.5 - 0.5 * s_, 0.0)
            vo = (o >> 7) * 512 + (o & 127)
            vecb[pl.ds(vo, _LANES)] = dx
            vecb[pl.ds(vo + 128, _LANES)] = dy
            vecb[pl.ds(vo + 256, _LANES)] = dz
            distb[pl.ds(o, _LANES)] = dist
            swb[pl.ds(o, _LANES)] = sw

        pltpu.sync_copy(vecb, vec_hbm.at[pl.ds(base * 4, _K * 4)])
        pltpu.sync_copy(distb, dist_hbm.at[pl.ds(base, _K)])
        pltpu.sync_copy(swb, sw_hbm.at[pl.ds(base, _K)])
        return carry

    lax.fori_loop(0, _NCH, chunk_z, None)


@functools.partial(jax.jit, donate_argnums=())
def _run(xcol, ycol, zcol, src, dst):
    mesh = plsc.VectorSubcoreMesh(core_axis_name="c", subcore_axis_name="s")
    f = pl.kernel(
        _edge_body,
        mesh=mesh,
        compiler_params=pltpu.CompilerParams(needs_layout_passes=False),
        out_type=(
            jax.ShapeDtypeStruct((_N_EDGES * 4,), jnp.float32),
            jax.ShapeDtypeStruct((_N_EDGES,), jnp.float32),
            jax.ShapeDtypeStruct((_N_EDGES,), jnp.float32),
            jax.ShapeDtypeStruct((_N_EDGES,), jnp.float32),
            jax.ShapeDtypeStruct((_N_EDGES,), jnp.float32),
        ),
        scratch_types=[
            pltpu.VMEM((_N_NODES,), jnp.float32),
            pltpu.VMEM((_K,), jnp.int32),
            pltpu.VMEM((_K,), jnp.int32),
            pltpu.VMEM((_K,), jnp.float32),
            pltpu.VMEM((_K,), jnp.float32),
            pltpu.VMEM((_K * 4,), jnp.float32),
            pltpu.VMEM((_K,), jnp.float32),
            pltpu.VMEM((_K,), jnp.float32),
            pltpu.SemaphoreType.DMA,
        ],
    )
    return f(xcol, ycol, zcol, src, dst)


def kernel(coordinates, edge_src, edge_dst):
    xcol = coordinates[:, 0]
    ycol = coordinates[:, 1]
    zcol = coordinates[:, 2]
    vecf, distances, switch, _, _ = _run(xcol, ycol, zcol, edge_src, edge_dst)
    vec = (vecf.reshape(_N_EDGES // 128, 4, 128)[:, :3, :]
           .transpose(0, 2, 1).reshape(_N_EDGES, 3))
    edge_mask = distances < _CUTOFF
    return (vec, distances, switch, edge_mask)


# speedup vs baseline: 57.9774x; 1.0357x over previous
import functools

import jax
import jax.numpy as jnp
from jax import lax
from jax.experimental import pallas as pl
from jax.experimental.pallas import tpu as pltpu
from jax.experimental.pallas import tpu_sc as plsc

_CUTOFF = 5.0
_N_NODES = 100000
_N_EDGES = 6400000

_LANES = 16
_NW = 32
_BLK = 128
_NBLK = _N_EDGES // _BLK
_BPT = _NBLK // _NW
_XTRA = _NBLK - _BPT * _NW
_BPC = 16
_K = _BPC * _BLK

_PI_OVER_CUTOFF = 0.6283185307179586
_HALF_PI = 1.5707963267948966


def _edge_body(x_hbm, y_hbm, z_hbm, src_hbm, dst_hbm,
               vec_hbm, dist_hbm, sw_hbm, dxp_hbm, dyp_hbm,
               table, srcb, dstb, dxb, dyb, vecb, distb, swb, sem):
    cid = lax.axis_index("c")
    sid = lax.axis_index("s")
    wid = sid * 2 + cid
    bstart = wid * _BPT + jnp.minimum(wid, _XTRA)
    bcnt = jnp.where(wid < _XTRA, _BPT + 1, _BPT)
    nch = (bcnt + _BPC - 1) // _BPC
    blast = bstart + bcnt - _BPC

    def chunk_base(s):
        return jnp.minimum(bstart + s * _BPC, blast) * _BLK

    def diff_pass(col_hbm, out_hbm):
        pltpu.sync_copy(col_hbm, table)

        def chunk(s, carry):
            base = chunk_base(s)
            pltpu.sync_copy(src_hbm.at[pl.ds(base, _K)], srcb)
            pltpu.sync_copy(dst_hbm.at[pl.ds(base, _K)], dstb)

            @plsc.parallel_loop(0, _K, step=_LANES, unroll=8)
            def inner(o):
                isrc = srcb[pl.ds(o, _LANES)]
                idst = dstb[pl.ds(o, _LANES)]
                cs = plsc.load_gather(table, [isrc])
                cd = plsc.load_gather(table, [idst])
                dxb[pl.ds(o, _LANES)] = cd - cs
            pltpu.sync_copy(dxb, out_hbm.at[pl.ds(base, _K)])
            return carry

        lax.fori_loop(0, nch, chunk, None)

    diff_pass(x_hbm, dxp_hbm)
    diff_pass(y_hbm, dyp_hbm)

    pltpu.sync_copy(z_hbm, table)

    def chunk_z(s, carry):
        base = chunk_base(s)
        pltpu.sync_copy(src_hbm.at[pl.ds(base, _K)], srcb)
        pltpu.sync_copy(dst_hbm.at[pl.ds(base, _K)], dstb)
        pltpu.sync_copy(dxp_hbm.at[pl.ds(base, _K)], dxb)
        pltpu.sync_copy(dyp_hbm.at[pl.ds(base, _K)], dyb)

        @plsc.parallel_loop(0, _K, step=_LANES, unroll=8)
        def inner(o):
            isrc = srcb[pl.ds(o, _LANES)]
            idst = dstb[pl.ds(o, _LANES)]
            zs = plsc.load_gather(table, [isrc])
            zd = plsc.load_gather(table, [idst])
            dz = zd - zs
            dx = dxb[pl.ds(o, _LANES)]
            dy = dyb[pl.ds(o, _LANES)]
            d2 = jnp.maximum(dx * dx + dy * dy + dz * dz, 1e-30)
            iy = jnp.int32(0x5F3759DF) - (plsc.bitcast(d2, jnp.int32) >> 1)
            y = plsc.bitcast(iy, jnp.float32)
            y = y * (1.5 - 0.5 * d2 * y * y)
            y = y * (1.5 - 0.5 * d2 * y * y)
            y = y * (1.5 - 0.5 * d2 * y * y)
            dist = d2 * y
            p = dist * _PI_OVER_CUTOFF - _HALF_PI
            z2 = p * p
            s_ = p * (1.0 + z2 * (-1.6666667e-01 + z2 * (8.3333333e-03
                 + z2 * (-1.9841270e-04 + z2 * 2.7557319e-06))))
            sw = jnp.where(dist < _CUTOFF, 0.5 - 0.5 * s_, 0.0)
            vo = (o >> 7) * 512 + (o & 127)
            vecb[pl.ds(vo, _LANES)] = dx
            vecb[pl.ds(vo + 128, _LANES)] = dy
            vecb[pl.ds(vo + 256, _LANES)] = dz
            distb[pl.ds(o, _LANES)] = dist
            swb[pl.ds(o, _LANES)] = sw

        pltpu.sync_copy(vecb, vec_hbm.at[pl.ds(base * 4, _K * 4)])
        pltpu.sync_copy(distb, dist_hbm.at[pl.ds(base, _K)])
        pltpu.sync_copy(swb, sw_hbm.at[pl.ds(base, _K)])
        return carry

    lax.fori_loop(0, nch, chunk_z, None)


@functools.partial(jax.jit, donate_argnums=())
def _run(xcol, ycol, zcol, src, dst):
    mesh = plsc.VectorSubcoreMesh(core_axis_name="c", subcore_axis_name="s")
    f = pl.kernel(
        _edge_body,
        mesh=mesh,
        compiler_params=pltpu.CompilerParams(needs_layout_passes=False),
        out_type=(
            jax.ShapeDtypeStruct((_N_EDGES * 4,), jnp.float32),
            jax.ShapeDtypeStruct((_N_EDGES,), jnp.float32),
            jax.ShapeDtypeStruct((_N_EDGES,), jnp.float32),
            jax.ShapeDtypeStruct((_N_EDGES,), jnp.float32),
            jax.ShapeDtypeStruct((_N_EDGES,), jnp.float32),
        ),
        scratch_types=[
            pltpu.VMEM((_N_NODES,), jnp.float32),
            pltpu.VMEM((_K,), jnp.int32),
            pltpu.VMEM((_K,), jnp.int32),
            pltpu.VMEM((_K,), jnp.float32),
            pltpu.VMEM((_K,), jnp.float32),
            pltpu.VMEM((_K * 4,), jnp.float32),
            pltpu.VMEM((_K,), jnp.float32),
            pltpu.VMEM((_K,), jnp.float32),
            pltpu.SemaphoreType.DMA,
        ],
    )
    return f(xcol, ycol, zcol, src, dst)


def kernel(coordinates, edge_src, edge_dst):
    xcol = coordinates[:, 0]
    ycol = coordinates[:, 1]
    zcol = coordinates[:, 2]
    vecf, distances, switch, _, _ = _run(xcol, ycol, zcol, edge_src, edge_dst)
    vec = (vecf.reshape(_N_EDGES // 128, 4, 128)[:, :3, :]
           .transpose(0, 2, 1).reshape(_N_EDGES, 3))
    edge_mask = distances < _CUTOFF
    return (vec, distances, switch, edge_mask)


# speedup vs baseline: 113.3410x; 1.9549x over previous
import functools

import jax
import jax.numpy as jnp
from jax import lax
from jax.experimental import pallas as pl
from jax.experimental.pallas import tpu as pltpu
from jax.experimental.pallas import tpu_sc as plsc

_CUTOFF = 5.0
_N_NODES = 100000
_N_EDGES = 6400000

_LANES = 16
_NW = 32
_BLK = 128
_NBLK = _N_EDGES // _BLK
_BPT = _NBLK // _NW
_XTRA = _NBLK - _BPT * _NW
_BPC = 8
_K = _BPC * _BLK
_NCH = -(-(_BPT + 1) // _BPC)

_PI_OVER_CUTOFF = 0.6283185307179586
_HALF_PI = 1.5707963267948966


def _edge_body(x_hbm, y_hbm, z_hbm, src_hbm, dst_hbm,
               vec_hbm, dist_hbm, sw_hbm, dxp_hbm, dyp_hbm,
               table, buf0, buf1, sem_in0, sem_in1, sem_out0, sem_out1):
    cid = lax.axis_index("c")
    sid = lax.axis_index("s")
    wid = sid * 2 + cid
    bstart = wid * _BPT + jnp.minimum(wid, _XTRA)
    bcnt = jnp.where(wid < _XTRA, _BPT + 1, _BPT)
    blast = bstart + bcnt - _BPC

    def chunk_base(s):
        return jnp.minimum(bstart + s * _BPC, blast) * _BLK

    bufs = (buf0, buf1)
    sems_in = (sem_in0, sem_in1)
    sems_out = (sem_out0, sem_out1)

    def run_pass(in_specs, out_specs, compute):
        def start_in(s, bi):
            base = chunk_base(s)
            for hbm, st, fi in in_specs:
                pltpu.async_copy(hbm.at[pl.ds(base * st, _K * st)],
                                 bufs[bi][fi], sems_in[bi])

        def wait_in(s, bi):
            base = chunk_base(s)
            for hbm, st, fi in in_specs:
                pltpu.make_async_copy(hbm.at[pl.ds(base * st, _K * st)],
                                      bufs[bi][fi], sems_in[bi]).wait()

        def start_out(s, bi):
            base = chunk_base(s)
            for hbm, st, fi in out_specs:
                pltpu.async_copy(bufs[bi][fi],
                                 hbm.at[pl.ds(base * st, _K * st)], sems_out[bi])

        def wait_out(s, bi):
            base = chunk_base(s)
            for hbm, st, fi in out_specs:
                pltpu.make_async_copy(bufs[bi][fi],
                                      hbm.at[pl.ds(base * st, _K * st)],
                                      sems_out[bi]).wait()

        start_in(0, 0)

        def pair(t, carry):
            c0 = 2 * t
            c1 = c0 + 1
            start_in(c1, 1)
            wait_in(c0, 0)

            @pl.when(t >= 1)
            def _():
                wait_out(c0, 0)

            compute(bufs[0])
            start_out(c0, 0)

            @pl.when(c0 + 2 < _NCH)
            def _():
                start_in(c0 + 2, 0)

            wait_in(c1, 1)

            @pl.when(t >= 1)
            def _():
                wait_out(c1, 1)

            compute(bufs[1])
            start_out(c1, 1)
            return carry

        lax.fori_loop(0, _NCH // 2, pair, None)
        wait_out(_NCH - 2, 0)
        wait_out(_NCH - 1, 1)

    def diff_compute(b):
        srcb, dstb, outb = b[0], b[1], b[2]

        @plsc.parallel_loop(0, _K, step=_LANES, unroll=8)
        def inner(o):
            isrc = srcb[pl.ds(o, _LANES)]
            idst = dstb[pl.ds(o, _LANES)]
            cs = plsc.load_gather(table, [isrc])
            cd = plsc.load_gather(table, [idst])
            outb[pl.ds(o, _LANES)] = cd - cs

    def z_compute(b):
        srcb, dstb, dxb, dyb, vecb, distb, swb = b

        @plsc.parallel_loop(0, _K, step=_LANES, unroll=8)
        def inner(o):
            isrc = srcb[pl.ds(o, _LANES)]
            idst = dstb[pl.ds(o, _LANES)]
            zs = plsc.load_gather(table, [isrc])
            zd = plsc.load_gather(table, [idst])
            dz = zd - zs
            dx = dxb[pl.ds(o, _LANES)]
            dy = dyb[pl.ds(o, _LANES)]
            d2 = jnp.maximum(dx * dx + dy * dy + dz * dz, 1e-30)
            iy = jnp.int32(0x5F3759DF) - (plsc.bitcast(d2, jnp.int32) >> 1)
            y = plsc.bitcast(iy, jnp.float32)
            y = y * (1.5 - 0.5 * d2 * y * y)
            y = y * (1.5 - 0.5 * d2 * y * y)
            y = y * (1.5 - 0.5 * d2 * y * y)
            dist = d2 * y
            p = dist * _PI_OVER_CUTOFF - _HALF_PI
            z2 = p * p
            s_ = p * (1.0 + z2 * (-1.6666667e-01 + z2 * (8.3333333e-03
                 + z2 * (-1.9841270e-04 + z2 * 2.7557319e-06))))
            sw = jnp.where(dist < _CUTOFF, 0.5 - 0.5 * s_, 0.0)
            vo = (o >> 7) * 512 + (o & 127)
            vecb[pl.ds(vo, _LANES)] = dx
            vecb[pl.ds(vo + 128, _LANES)] = dy
            vecb[pl.ds(vo + 256, _LANES)] = dz
            distb[pl.ds(o, _LANES)] = dist
            swb[pl.ds(o, _LANES)] = sw

    pltpu.sync_copy(x_hbm, table)
    run_pass([(src_hbm, 1, 0), (dst_hbm, 1, 1)], [(dxp_hbm, 1, 2)],
             diff_compute)
    pltpu.sync_copy(y_hbm, table)
    run_pass([(src_hbm, 1, 0), (dst_hbm, 1, 1)], [(dyp_hbm, 1, 2)],
             diff_compute)
    pltpu.sync_copy(z_hbm, table)
    run_pass([(src_hbm, 1, 0), (dst_hbm, 1, 1), (dxp_hbm, 1, 2),
              (dyp_hbm, 1, 3)],
             [(vec_hbm, 4, 4), (dist_hbm, 1, 5), (sw_hbm, 1, 6)],
             z_compute)


@functools.partial(jax.jit, donate_argnums=())
def _run(xcol, ycol, zcol, src, dst):
    mesh = plsc.VectorSubcoreMesh(core_axis_name="c", subcore_axis_name="s")
    bufset = (
        pltpu.VMEM((_K,), jnp.int32),
        pltpu.VMEM((_K,), jnp.int32),
        pltpu.VMEM((_K,), jnp.float32),
        pltpu.VMEM((_K,), jnp.float32),
        pltpu.VMEM((_K * 4,), jnp.float32),
        pltpu.VMEM((_K,), jnp.float32),
        pltpu.VMEM((_K,), jnp.float32),
    )
    f = pl.kernel(
        _edge_body,
        mesh=mesh,
        compiler_params=pltpu.CompilerParams(needs_layout_passes=False),
        out_type=(
            jax.ShapeDtypeStruct((_N_EDGES * 4,), jnp.float32),
            jax.ShapeDtypeStruct((_N_EDGES,), jnp.float32),
            jax.ShapeDtypeStruct((_N_EDGES,), jnp.float32),
            jax.ShapeDtypeStruct((_N_EDGES,), jnp.float32),
            jax.ShapeDtypeStruct((_N_EDGES,), jnp.float32),
        ),
        scratch_types=[
            pltpu.VMEM((_N_NODES,), jnp.float32),
            bufset,
            bufset,
            pltpu.SemaphoreType.DMA,
            pltpu.SemaphoreType.DMA,
            pltpu.SemaphoreType.DMA,
            pltpu.SemaphoreType.DMA,
        ],
    )
    return f(xcol, ycol, zcol, src, dst)


def kernel(coordinates, edge_src, edge_dst):
    xcol = coordinates[:, 0]
    ycol = coordinates[:, 1]
    zcol = coordinates[:, 2]
    vecf, distances, switch, _, _ = _run(xcol, ycol, zcol, edge_src, edge_dst)
    vec = (vecf.reshape(_N_EDGES // 128, 4, 128)[:, :3, :]
           .transpose(0, 2, 1).reshape(_N_EDGES, 3))
    edge_mask = distances < _CUTOFF
    return (vec, distances, switch, edge_mask)


# speedup vs baseline: 129.5656x; 1.1431x over previous
import functools

import jax
import jax.numpy as jnp
from jax import lax
from jax.experimental import pallas as pl
from jax.experimental.pallas import tpu as pltpu
from jax.experimental.pallas import tpu_sc as plsc

_CUTOFF = 5.0
_N_NODES = 100000
_N_EDGES = 6400000

_LANES = 16
_NW = 32
_BLK = 128
_NBLK = _N_EDGES // _BLK
_BPT = _NBLK // _NW
_XTRA = _NBLK - _BPT * _NW
_BPC = 12
_K = _BPC * _BLK
_NCH = -(-(_BPT + 1) // _BPC)

_PI_OVER_CUTOFF = 0.6283185307179586
_HALF_PI = 1.5707963267948966


def _edge_body(x_hbm, y_hbm, z_hbm, src_hbm, dst_hbm,
               vec_hbm, dist_hbm, sw_hbm, dxp_hbm, dyp_hbm,
               table, buf0, buf1, sem_in0, sem_in1, sem_out0, sem_out1):
    cid = lax.axis_index("c")
    sid = lax.axis_index("s")
    wid = sid * 2 + cid
    bstart = wid * _BPT + jnp.minimum(wid, _XTRA)
    bcnt = jnp.where(wid < _XTRA, _BPT + 1, _BPT)
    blast = bstart + bcnt - _BPC

    def chunk_base(s):
        return jnp.minimum(bstart + s * _BPC, blast) * _BLK

    bufs = (buf0, buf1)
    sems_in = (sem_in0, sem_in1)
    sems_out = (sem_out0, sem_out1)

    def run_pass(in_specs, out_specs, compute):
        def start_in(s, bi):
            base = chunk_base(s)
            for hbm, st, fi in in_specs:
                pltpu.async_copy(hbm.at[pl.ds(base * st, _K * st)],
                                 bufs[bi][fi], sems_in[bi])

        def wait_in(s, bi):
            base = chunk_base(s)
            for hbm, st, fi in in_specs:
                pltpu.make_async_copy(hbm.at[pl.ds(base * st, _K * st)],
                                      bufs[bi][fi], sems_in[bi]).wait()

        def start_out(s, bi):
            base = chunk_base(s)
            for hbm, st, fi in out_specs:
                pltpu.async_copy(bufs[bi][fi],
                                 hbm.at[pl.ds(base * st, _K * st)], sems_out[bi])

        def wait_out(s, bi):
            base = chunk_base(s)
            for hbm, st, fi in out_specs:
                pltpu.make_async_copy(bufs[bi][fi],
                                      hbm.at[pl.ds(base * st, _K * st)],
                                      sems_out[bi]).wait()

        start_in(0, 0)

        def pair(t, carry):
            c0 = 2 * t
            c1 = c0 + 1
            start_in(c1, 1)
            wait_in(c0, 0)

            @pl.when(t >= 1)
            def _():
                wait_out(c0, 0)

            compute(bufs[0])
            start_out(c0, 0)

            @pl.when(c0 + 2 < _NCH)
            def _():
                start_in(c0 + 2, 0)

            wait_in(c1, 1)

            @pl.when(t >= 1)
            def _():
                wait_out(c1, 1)

            compute(bufs[1])
            start_out(c1, 1)
            return carry

        lax.fori_loop(0, _NCH // 2, pair, None)
        if _NCH % 2:
            c = _NCH - 1
            wait_in(c, 0)
            wait_out(c - 2, 0)
            compute(bufs[0])
            start_out(c, 0)
            wait_out(c, 0)
            wait_out(c - 1, 1)
        else:
            wait_out(_NCH - 2, 0)
            wait_out(_NCH - 1, 1)

    def diff_compute(b):
        srcb, dstb, outb = b[0], b[1], b[2]

        @plsc.parallel_loop(0, _K, step=_LANES, unroll=8)
        def inner(o):
            isrc = srcb[pl.ds(o, _LANES)]
            idst = dstb[pl.ds(o, _LANES)]
            cs = plsc.load_gather(table, [isrc])
            cd = plsc.load_gather(table, [idst])
            outb[pl.ds(o, _LANES)] = cd - cs

    def z_compute(b):
        srcb, dstb, dxb, dyb, vecb, distb, swb = b

        @plsc.parallel_loop(0, _K, step=_LANES, unroll=8)
        def inner(o):
            isrc = srcb[pl.ds(o, _LANES)]
            idst = dstb[pl.ds(o, _LANES)]
            zs = plsc.load_gather(table, [isrc])
            zd = plsc.load_gather(table, [idst])
            dz = zd - zs
            dx = dxb[pl.ds(o, _LANES)]
            dy = dyb[pl.ds(o, _LANES)]
            d2 = jnp.maximum(dx * dx + dy * dy + dz * dz, 1e-30)
            iy = jnp.int32(0x5F3759DF) - (plsc.bitcast(d2, jnp.int32) >> 1)
            y = plsc.bitcast(iy, jnp.float32)
            y = y * (1.5 - 0.5 * d2 * y * y)
            y = y * (1.5 - 0.5 * d2 * y * y)
            y = y * (1.5 - 0.5 * d2 * y * y)
            dist = d2 * y
            p = dist * _PI_OVER_CUTOFF - _HALF_PI
            z2 = p * p
            s_ = p * (1.0 + z2 * (-1.6666667e-01 + z2 * (8.3333333e-03
                 + z2 * (-1.9841270e-04 + z2 * 2.7557319e-06))))
            sw = jnp.where(dist < _CUTOFF, 0.5 - 0.5 * s_, 0.0)
            vo = (o >> 7) * 512 + (o & 127)
            vecb[pl.ds(vo, _LANES)] = dx
            vecb[pl.ds(vo + 128, _LANES)] = dy
            vecb[pl.ds(vo + 256, _LANES)] = dz
            distb[pl.ds(o, _LANES)] = dist
            swb[pl.ds(o, _LANES)] = sw

    pltpu.sync_copy(x_hbm, table)
    run_pass([(src_hbm, 1, 0), (dst_hbm, 1, 1)], [(dxp_hbm, 1, 2)],
             diff_compute)
    pltpu.sync_copy(y_hbm, table)
    run_pass([(src_hbm, 1, 0), (dst_hbm, 1, 1)], [(dyp_hbm, 1, 2)],
             diff_compute)
    pltpu.sync_copy(z_hbm, table)
    run_pass([(src_hbm, 1, 0), (dst_hbm, 1, 1), (dxp_hbm, 1, 2),
              (dyp_hbm, 1, 3)],
             [(vec_hbm, 4, 4), (dist_hbm, 1, 5), (sw_hbm, 1, 6)],
             z_compute)


@functools.partial(jax.jit, donate_argnums=())
def _run(xcol, ycol, zcol, src, dst):
    mesh = plsc.VectorSubcoreMesh(core_axis_name="c", subcore_axis_name="s")
    bufset = (
        pltpu.VMEM((_K,), jnp.int32),
        pltpu.VMEM((_K,), jnp.int32),
        pltpu.VMEM((_K,), jnp.float32),
        pltpu.VMEM((_K,), jnp.float32),
        pltpu.VMEM((_K * 4,), jnp.float32),
        pltpu.VMEM((_K,), jnp.float32),
        pltpu.VMEM((_K,), jnp.float32),
    )
    f = pl.kernel(
        _edge_body,
        mesh=mesh,
        compiler_params=pltpu.CompilerParams(needs_layout_passes=False),
        out_type=(
            jax.ShapeDtypeStruct((_N_EDGES * 4,), jnp.float32),
            jax.ShapeDtypeStruct((_N_EDGES,), jnp.float32),
            jax.ShapeDtypeStruct((_N_EDGES,), jnp.float32),
            jax.ShapeDtypeStruct((_N_EDGES,), jnp.float32),
            jax.ShapeDtypeStruct((_N_EDGES,), jnp.float32),
        ),
        scratch_types=[
            pltpu.VMEM((_N_NODES,), jnp.float32),
            bufset,
            bufset,
            pltpu.SemaphoreType.DMA,
            pltpu.SemaphoreType.DMA,
            pltpu.SemaphoreType.DMA,
            pltpu.SemaphoreType.DMA,
        ],
    )
    return f(xcol, ycol, zcol, src, dst)


def kernel(coordinates, edge_src, edge_dst):
    xcol = coordinates[:, 0]
    ycol = coordinates[:, 1]
    zcol = coordinates[:, 2]
    vecf, distances, switch, _, _ = _run(xcol, ycol, zcol, edge_src, edge_dst)
    vec = (vecf.reshape(_N_EDGES // 128, 4, 128)[:, :3, :]
           .transpose(0, 2, 1).reshape(_N_EDGES, 3))
    edge_mask = distances < _CUTOFF
    return (vec, distances, switch, edge_mask)
